# Initial kernel scaffold; baseline (speedup 1.0000x reference)
import jax
import jax.numpy as jnp
from jax.experimental import pallas as pl


def kernel(features, type_mask, edge_metapath_indices_0, edge_metapath_indices_1, dst_0, dst_1, r_vec, attn_0, attn_1, fc1_w, fc1_b, fc2_w, fc_w, fc_b):
    raise NotImplementedError("write your pallas kernel here")


# speedup vs baseline: 2.7375x; 2.7375x over previous
import functools
import jax
import jax.numpy as jnp
from jax.experimental import pallas as pl

N_NODES = 10000
IN_DIM = 128
OUT_DIM = 128
NUM_HEADS = 8
E_MP = 160000
L = 3
_ETYPES = [[0, 1], [1, 0]]


def _complex_mul(ar, ai, br, bi):
    return ar * br - ai * bi, ar * bi + ai * br


def _rot_tables(features, r_vec, etypes):
    rv = r_vec / (jnp.linalg.norm(r_vec, axis=2, keepdims=True) + 1e-12)
    fr_r = [None] * L
    fr_i = [None] * L
    fr_r[L - 1] = jnp.ones((IN_DIM // 2,), jnp.float32)
    fr_i[L - 1] = jnp.zeros((IN_DIM // 2,), jnp.float32)
    for i in range(L - 2, -1, -1):
        fr_r[i], fr_i[i] = _complex_mul(
            fr_r[i + 1], fr_i[i + 1], rv[etypes[i], :, 0], rv[etypes[i], :, 1])
    f = features.reshape(N_NODES, IN_DIM // 2, 2)
    tabs = []
    for i in range(L - 1):
        re, im = _complex_mul(f[:, :, 0], f[:, :, 1], fr_r[i], fr_i[i])
        tabs.append(jnp.stack([re, im], axis=-1).reshape(N_NODES, IN_DIM))
    tabs.append(features)
    return tabs


def _metapath_jnp(tabs, indices, dst, attn):
    hidden = (tabs[0][indices[:, 0]] + tabs[1][indices[:, 1]]
              + tabs[2][indices[:, 2]]) * (1.0 / 3.0)
    a = hidden @ attn.T
    a = jnp.where(a > 0, a, 0.01 * a)
    ae = jnp.exp(a)
    asum = jax.ops.segment_sum(ae, dst, num_segments=N_NODES)
    w = ae / (asum[dst] + 1e-16)
    ft = jax.ops.segment_sum(w[:, :, None] * hidden[:, None, :], dst,
                             num_segments=N_NODES)
    return ft.reshape(N_NODES, NUM_HEADS * IN_DIM)


def _final_kernel(o0_ref, o1_ref, beta_ref, fcw_ref, fcb_ref, h_ref, hfc_ref):
    b0 = beta_ref[0, 0]
    b1 = beta_ref[0, 1]
    h = b0 * o0_ref[...] + b1 * o1_ref[...]
    h_ref[...] = h
    hfc_ref[...] = jax.lax.dot_general(
        h, fcw_ref[...], (((1,), (1,)), ((), ())),
        preferred_element_type=jnp.float32) + fcb_ref[...][None, :]


def kernel(features, type_mask, edge_metapath_indices_0,
           edge_metapath_indices_1, dst_0, dst_1, r_vec, attn_0, attn_1,
           fc1_w, fc1_b, fc2_w, fc_w, fc_b):
    outs = []
    for idx, dst, et, attn in [
        (edge_metapath_indices_0, dst_0, _ETYPES[0], attn_0),
        (edge_metapath_indices_1, dst_1, _ETYPES[1], attn_1),
    ]:
        tabs = _rot_tables(features, r_vec, et)
        o = _metapath_jnp(tabs, idx, dst, attn)
        outs.append(jax.nn.elu(o))
    betas = []
    for o in outs:
        f1 = jnp.tanh(o @ fc1_w.T + fc1_b)
        betas.append((fc2_w @ f1.mean(axis=0))[0])
    beta = jax.nn.softmax(jnp.stack(betas))

    blk = 1000
    grid = N_NODES // blk
    h_fc, h = pl.pallas_call(
        _final_kernel,
        grid=(grid,),
        in_specs=[
            pl.BlockSpec((blk, NUM_HEADS * IN_DIM), lambda i: (i, 0)),
            pl.BlockSpec((blk, NUM_HEADS * IN_DIM), lambda i: (i, 0)),
            pl.BlockSpec((1, 2), lambda i: (0, 0)),
            pl.BlockSpec((OUT_DIM, NUM_HEADS * IN_DIM), lambda i: (0, 0)),
            pl.BlockSpec((OUT_DIM,), lambda i: (0,)),
        ],
        out_specs=[
            pl.BlockSpec((blk, NUM_HEADS * IN_DIM), lambda i: (i, 0)),
            pl.BlockSpec((blk, OUT_DIM), lambda i: (i, 0)),
        ],
        out_shape=[
            jax.ShapeDtypeStruct((N_NODES, NUM_HEADS * IN_DIM), jnp.float32),
            jax.ShapeDtypeStruct((N_NODES, OUT_DIM), jnp.float32),
        ],
    )(outs[0], outs[1], beta.reshape(1, 2), fc_w, fc_b)[::-1]
    return (h_fc, h)


# speedup vs baseline: 6.3652x; 2.3251x over previous
import functools
import jax
import jax.numpy as jnp
from jax import lax
from jax.experimental import pallas as pl
from jax.experimental.pallas import tpu as pltpu
from jax.experimental.pallas import tpu_sc as plsc

N_NODES = 10000
IN_DIM = 128
OUT_DIM = 128
NUM_HEADS = 8
E_MP = 160000
L = 3

NW = 32
LN = 16
EPW = E_MP // NW
C1 = 40
NCH1 = EPW // C1
G1 = 3
RECR = E_MP // 8

NN = 96
ROUNDS = 4
NPAD = NN * NW * ROUNDS
SC_C = 3200
NSCCH = E_MP // SC_C
SUBC = 32
ACC_W = NN * NUM_HEADS * IN_DIM


def _iota16():
    return lax.broadcasted_iota(jnp.int32, (LN,), 0)


def _splat(x):
    return jnp.full((LN,), x, jnp.int32)


def _tables_kernel(f_ref, m_ref, ms_ref, fall_ref, sall_ref):
    f = f_ref[...]
    for v in range(4):
        fall_ref[v] = jax.lax.dot_general(
            f, m_ref[v], (((1,), (0,)), ((), ())),
            preferred_element_type=jnp.float32)
        sall_ref[v] = jax.lax.dot_general(
            f, ms_ref[v], (((1,), (0,)), ((), ())),
            preferred_element_type=jnp.float32)


def _build_tables(features, m, ms):
    blk = 1000
    grid = N_NODES // blk
    return pl.pallas_call(
        _tables_kernel,
        grid=(grid,),
        in_specs=[
            pl.BlockSpec((blk, IN_DIM), lambda i: (i, 0)),
            pl.BlockSpec((4, IN_DIM, IN_DIM), lambda i: (0, 0, 0)),
            pl.BlockSpec((4, IN_DIM, IN_DIM), lambda i: (0, 0, 0)),
        ],
        out_specs=[
            pl.BlockSpec((4, blk, IN_DIM), lambda i: (0, i, 0)),
            pl.BlockSpec((4, blk, IN_DIM), lambda i: (0, i, 0)),
        ],
        out_shape=[
            jax.ShapeDtypeStruct((4, N_NODES, IN_DIM), jnp.float32),
            jax.ShapeDtypeStruct((4, N_NODES, IN_DIM), jnp.float32),
        ],
    )(features, m, ms)


def _pass1_body(off0, off1, off2, h_off,
                pk_hbm, s_hbm,
                rec_hbm, asum_hbm,
                rowb, pkb, fidx, srows, recs, asuml, sem):
    wid = lax.axis_index("s") * 2 + lax.axis_index("c")
    iota = _iota16()
    zf = jnp.zeros((LN,), jnp.float32)
    offs = (off0, off1, off2)
    nrow = C1 // 8

    def init_asum(k, _):
        plsc.store_scatter(asuml, [iota + k * LN], zf)
        return 0
    lax.fori_loop(0, N_NODES * NUM_HEADS // LN, init_asum, 0)

    def chunk(c, _):
        e0 = wid * EPW + c * C1
        mk5 = iota < nrow
        plsc.store_scatter(rowb, [iota], (e0 // 8) + iota, mask=mk5)
        pltpu.async_copy(pk_hbm.at[rowb], pkb, sem).wait()

        def build(g, _):
            eloc = g * LN + iota
            mk = eloc < C1
            row = lax.shift_right_logical(eloc, 3)
            col0 = lax.bitwise_and(eloc, 7) * 16
            dstv = plsc.load_gather(pkb, [row, col0 + 3], mask=mk)
            plsc.store_scatter(recs, [iota * 16 + (g * 256 + 3)], dstv,
                               mask=mk)
            for t in range(L):
                it = plsc.load_gather(pkb, [row, col0 + t], mask=mk)
                ft = it + offs[t]
                plsc.store_scatter(fidx, [iota * 3 + (g * 48 + t)], ft,
                                   mask=mk)
                plsc.store_scatter(recs, [iota * 16 + (g * 256 + t)],
                                   ft, mask=mk)
            return 0
        lax.fori_loop(0, G1, build, 0)

        pltpu.async_copy(s_hbm.at[fidx], srows, sem).wait()

        def heads(g, _):
            mk = (g * LN + iota) < C1
            dstv = plsc.load_gather(recs, [iota * 16 + (g * 256 + 3)],
                                    mask=mk)
            for h in range(NUM_HEADS):
                col = h_off + h
                a = zf
                for t in range(L):
                    a = a + plsc.load_gather(
                        srows, [iota * 3 + (g * 48 + t), _splat(col)],
                        mask=mk)
                a = a * (1.0 / 3.0)
                a = jnp.maximum(a, 0.01 * a)
                ae = jnp.exp(a)
                plsc.addupdate_scatter(asuml, [dstv * NUM_HEADS + h], ae,
                                       mask=mk)
                plsc.store_scatter(recs,
                                   [iota * 16 + (g * 256 + 4 + h)],
                                   plsc.bitcast(ae, jnp.int32), mask=mk)
            return 0
        lax.fori_loop(0, G1, heads, 0)

        pltpu.sync_copy(recs, rec_hbm.at[pl.ds(e0 * 16, C1 * 16)])
        return 0
    lax.fori_loop(0, NCH1, chunk, 0)

    pltpu.sync_copy(asuml,
                    asum_hbm.at[pl.ds(wid * N_NODES * NUM_HEADS,
                                      N_NODES * NUM_HEADS)])


def _make_pass1(off0, off1, off2, h_off):
    mesh = plsc.VectorSubcoreMesh(core_axis_name="c", subcore_axis_name="s")
    return pl.kernel(
        functools.partial(_pass1_body, off0, off1, off2, h_off),
        out_type=[
            jax.ShapeDtypeStruct((E_MP * 16,), jnp.int32),
            jax.ShapeDtypeStruct((NW * N_NODES * NUM_HEADS,), jnp.float32),
        ],
        mesh=mesh,
        compiler_params=pltpu.CompilerParams(needs_layout_passes=False),
        scratch_types=[
            pltpu.VMEM((C1 // 8,), jnp.int32),
            pltpu.VMEM((C1 // 8, 128), jnp.int32),
            pltpu.VMEM((C1 * L,), jnp.int32),
            pltpu.VMEM((C1 * L, IN_DIM), jnp.float32),
            pltpu.VMEM((C1 * 16,), jnp.int32),
            pltpu.VMEM((N_NODES * NUM_HEADS,), jnp.float32),
            pltpu.SemaphoreType.DMA,
        ],
    )


def _pass2_body(rec_hbm, dst_hbm, asum_hbm, f_hbm, out_hbm,
                dstbuf, hits, rowbuf, recb, fidx2, frows, asumo, acc, sem):
    wid = lax.axis_index("s") * 2 + lax.axis_index("c")
    iota = _iota16()
    zf = jnp.zeros((LN,), jnp.float32)
    zi = jnp.zeros((LN,), jnp.int32)

    for k in range(SC_C // LN):
        plsc.store_scatter(hits, [iota + k * LN], zi)

    def round_body(r, _):
        nbase = (r * NW + wid) * NN

        def init_acc(k, _):
            plsc.store_scatter(acc, [iota + k * LN], zf)
            return 0
        lax.fori_loop(0, ACC_W // LN, init_acc, 0)

        pltpu.sync_copy(asum_hbm.at[pl.ds(nbase * NUM_HEADS, NN * NUM_HEADS)],
                        asumo)

        def chunk(c, _):
            e0 = c * SC_C
            pltpu.sync_copy(dst_hbm.at[pl.ds(e0, SC_C)], dstbuf)

            def scan(g, nh):
                dv = plsc.load_gather(dstbuf, [iota + g * LN])
                mk = (dv >= nbase) & (dv < nbase + NN)
                mi = mk.astype(jnp.int32)
                pos = jnp.maximum(nh + plsc.cumsum(mi) - 1, 0)
                eid = e0 + g * LN + iota
                plsc.store_scatter(hits, [pos], eid, mask=mk)
                return nh + jnp.sum(mi)
            nh = lax.fori_loop(0, SC_C // LN, scan, jnp.int32(0))

            def sub(j, _):
                j0 = j * SUBC
                for g3 in range(SUBC // LN):
                    ej = plsc.load_gather(hits, [j0 + g3 * LN + iota])
                    plsc.store_scatter(rowbuf, [g3 * LN + iota],
                                       lax.shift_right_logical(ej, 3))
                pltpu.async_copy(rec_hbm.at[rowbuf], recb, sem).wait()
                for g3 in range(SUBC // LN):
                    ej = plsc.load_gather(hits, [j0 + g3 * LN + iota])
                    sub16 = lax.bitwise_and(ej, 7) * 16
                    for t in range(L):
                        ft = plsc.load_gather(
                            recb, [g3 * LN + iota, sub16 + t])
                        plsc.store_scatter(
                            fidx2, [iota * 3 + (g3 * 48 + t)], ft)
                pltpu.async_copy(f_hbm.at[fidx2], frows, sem).wait()
                for g3 in range(SUBC // LN):
                    lanepos = j0 + g3 * LN + iota
                    mk2 = lanepos < nh
                    ej = plsc.load_gather(hits, [j0 + g3 * LN + iota])
                    sub16 = lax.bitwise_and(ej, 7) * 16
                    dv2 = plsc.load_gather(recb, [g3 * LN + iota, sub16 + 3])
                    dstl = jnp.clip(dv2 - nbase, 0, NN - 1)
                    rowbase = dstl * (NUM_HEADS * IN_DIM)
                    ws = []
                    for h in range(NUM_HEADS):
                        aeh = plsc.bitcast(
                            plsc.load_gather(
                                recb, [g3 * LN + iota, sub16 + 4 + h]),
                            jnp.float32)
                        ash = plsc.load_gather(asumo,
                                               [dstl * NUM_HEADS + h])
                        ws.append(aeh / (ash + 1e-16))

                    def dloop(d, _):
                        hd = zf
                        for t in range(L):
                            hd = hd + plsc.load_gather(
                                frows,
                                [iota * 3 + (g3 * 48 + t), _splat(d)])
                        hd = hd * (1.0 / 3.0)
                        for h in range(NUM_HEADS):
                            plsc.addupdate_scatter(
                                acc, [rowbase + (h * IN_DIM + d)],
                                ws[h] * hd, mask=mk2)
                        return 0
                    lax.fori_loop(0, IN_DIM, dloop, 0)
                return 0
            lax.fori_loop(0, (nh + SUBC - 1) // SUBC, sub, 0)
            return 0
        lax.fori_loop(0, NSCCH, chunk, 0)

        pltpu.sync_copy(acc, out_hbm.at[pl.ds(nbase * NUM_HEADS * IN_DIM,
                                              ACC_W)])
        return 0
    lax.fori_loop(0, ROUNDS, round_body, 0)


def _make_pass2():
    mesh = plsc.VectorSubcoreMesh(core_axis_name="c", subcore_axis_name="s")
    return pl.kernel(
        _pass2_body,
        out_type=jax.ShapeDtypeStruct((NPAD * NUM_HEADS * IN_DIM,),
                                      jnp.float32),
        mesh=mesh,
        compiler_params=pltpu.CompilerParams(needs_layout_passes=False),
        scratch_types=[
            pltpu.VMEM((SC_C,), jnp.int32),
            pltpu.VMEM((SC_C,), jnp.int32),
            pltpu.VMEM((SUBC,), jnp.int32),
            pltpu.VMEM((SUBC, 128), jnp.int32),
            pltpu.VMEM((SUBC * L,), jnp.int32),
            pltpu.VMEM((SUBC * L, IN_DIM), jnp.float32),
            pltpu.VMEM((NN * NUM_HEADS,), jnp.float32),
            pltpu.VMEM((ACC_W,), jnp.float32),
            pltpu.SemaphoreType.DMA,
        ],
    )


def _b1_kernel(ft0_ref, ft1_ref, w_ref, b_ref, o0_ref, o1_ref, p_ref):
    for i, (ft_ref, o_ref) in enumerate([(ft0_ref, o0_ref),
                                         (ft1_ref, o1_ref)]):
        x = ft_ref[...]
        o = jnp.where(x > 0, x, jnp.exp(jnp.minimum(x, 0.0)) - 1.0)
        o_ref[...] = o
        f1 = jnp.tanh(jax.lax.dot_general(
            o, w_ref[...], (((1,), (1,)), ((), ())),
            preferred_element_type=jnp.float32) + b_ref[...][None, :])
        p_ref[0, i] = jnp.sum(f1, axis=0)


def _run_b1(ft0, ft1, fc1_w, fc1_b):
    blk = 1000
    grid = N_NODES // blk
    dh = NUM_HEADS * IN_DIM
    return pl.pallas_call(
        _b1_kernel,
        grid=(grid,),
        in_specs=[
            pl.BlockSpec((blk, dh), lambda i: (i, 0)),
            pl.BlockSpec((blk, dh), lambda i: (i, 0)),
            pl.BlockSpec((128, dh), lambda i: (0, 0)),
            pl.BlockSpec((128,), lambda i: (0,)),
        ],
        out_specs=[
            pl.BlockSpec((blk, dh), lambda i: (i, 0)),
            pl.BlockSpec((blk, dh), lambda i: (i, 0)),
            pl.BlockSpec((1, 2, 128), lambda i: (i, 0, 0)),
        ],
        out_shape=[
            jax.ShapeDtypeStruct((N_NODES, dh), jnp.float32),
            jax.ShapeDtypeStruct((N_NODES, dh), jnp.float32),
            jax.ShapeDtypeStruct((grid, 2, 128), jnp.float32),
        ],
    )(ft0, ft1, fc1_w, fc1_b)


def _final_kernel(o0_ref, o1_ref, beta_ref, fcw_ref, fcb_ref, h_ref, hfc_ref):
    b0 = beta_ref[0, 0]
    b1 = beta_ref[0, 1]
    h = b0 * o0_ref[...] + b1 * o1_ref[...]
    h_ref[...] = h
    hfc_ref[...] = jax.lax.dot_general(
        h, fcw_ref[...], (((1,), (1,)), ((), ())),
        preferred_element_type=jnp.float32) + fcb_ref[...][None, :]


def _run_final(o0, o1, beta, fc_w, fc_b):
    blk = 1000
    grid = N_NODES // blk
    dh = NUM_HEADS * IN_DIM
    h, h_fc = pl.pallas_call(
        _final_kernel,
        grid=(grid,),
        in_specs=[
            pl.BlockSpec((blk, dh), lambda i: (i, 0)),
            pl.BlockSpec((blk, dh), lambda i: (i, 0)),
            pl.BlockSpec((1, 2), lambda i: (0, 0)),
            pl.BlockSpec((OUT_DIM, dh), lambda i: (0, 0)),
            pl.BlockSpec((OUT_DIM,), lambda i: (0,)),
        ],
        out_specs=[
            pl.BlockSpec((blk, dh), lambda i: (i, 0)),
            pl.BlockSpec((blk, OUT_DIM), lambda i: (i, 0)),
        ],
        out_shape=[
            jax.ShapeDtypeStruct((N_NODES, dh), jnp.float32),
            jax.ShapeDtypeStruct((N_NODES, OUT_DIM), jnp.float32),
        ],
    )(o0, o1, beta.reshape(1, 2), fc_w, fc_b)
    return h, h_fc


def _rot_matrices(r_vec):
    rv = r_vec / (jnp.linalg.norm(r_vec, axis=2, keepdims=True) + 1e-12)
    r0re, r0im = rv[0, :, 0], rv[0, :, 1]
    r1re, r1im = rv[1, :, 0], rv[1, :, 1]
    p_re = r0re * r1re - r0im * r1im
    p_im = r0re * r1im + r0im * r1re
    ident = (jnp.ones(IN_DIM // 2), jnp.zeros(IN_DIM // 2))
    eye = jnp.eye(IN_DIM // 2, dtype=jnp.float32)
    mats = []
    for re, im in [(p_re, p_im), (r1re, r1im), (r0re, r0im), ident]:
        r2 = jnp.stack([jnp.stack([re, im]), jnp.stack([-im, re])])
        mats.append(jnp.einsum('kl,abk->kalb', eye, r2).reshape(IN_DIM,
                                                                IN_DIM))
    return jnp.stack(mats)


def kernel(features, type_mask, edge_metapath_indices_0,
           edge_metapath_indices_1, dst_0, dst_1, r_vec, attn_0, attn_1,
           fc1_w, fc1_b, fc2_w, fc_w, fc_b):
    attn_cat = jnp.concatenate([attn_0, attn_1], axis=0)
    m = _rot_matrices(r_vec)
    ms_pad = jnp.pad(jnp.einsum('vij,hj->vih', m, attn_cat),
                     ((0, 0), (0, 0), (0, IN_DIM - 16)))
    f_all, s_all = _build_tables(features, m, ms_pad)
    f_flat = f_all.reshape(4 * N_NODES, IN_DIM)
    s_flat = s_all.reshape(4 * N_NODES, IN_DIM)

    pass1s = [_make_pass1(0, N_NODES, 3 * N_NODES, 0),
              _make_pass1(0, 2 * N_NODES, 3 * N_NODES, 8)]
    pass2 = _make_pass2()
    fts = []
    for mp, (idx, dst) in enumerate([(edge_metapath_indices_0, dst_0),
                                     (edge_metapath_indices_1, dst_1)]):
        idx32 = idx.astype(jnp.int32)
        dst32 = dst.astype(jnp.int32)
        pk = jnp.concatenate(
            [idx32, dst32[:, None],
             jnp.zeros((E_MP, 12), jnp.int32)], axis=1).reshape(RECR, 128)
        rec, parts = pass1s[mp](pk, s_flat)
        asum = parts.reshape(NW, N_NODES * NUM_HEADS).sum(axis=0)
        asum_pad = jnp.pad(asum, (0, (NPAD - N_NODES) * NUM_HEADS))
        ftflat = pass2(rec.reshape(RECR, 128), dst32, asum_pad, f_flat)
        fts.append(ftflat.reshape(NPAD, NUM_HEADS * IN_DIM)[:N_NODES])

    o0, o1, p = _run_b1(fts[0], fts[1], fc1_w, fc1_b)
    f1m = p.sum(axis=0) * (1.0 / N_NODES)
    betas = f1m @ fc2_w[0]
    beta = jax.nn.softmax(betas)
    h, h_fc = _run_final(o0, o1, beta, fc_w, fc_b)
    return (h_fc, h)


# speedup vs baseline: 14.4397x; 2.2685x over previous
import functools
import jax
import jax.numpy as jnp
from jax import lax
from jax.experimental import pallas as pl
from jax.experimental.pallas import tpu as pltpu
from jax.experimental.pallas import tpu_sc as plsc

N_NODES = 10000
IN_DIM = 128
OUT_DIM = 128
NUM_HEADS = 8
E_MP = 160000
L = 3

NW = 32
LN = 16
EPW = E_MP // NW
C1 = 40
NCH1 = EPW // C1
G1 = 3
RECR = E_MP // 8

NN = 96
ROUNDS = 4
NPAD = NN * NW * ROUNDS
SC_C = 3200
NSCCH = E_MP // SC_C
SUBC = 16
ACC_W = NN * NUM_HEADS * IN_DIM


def _iota16():
    return lax.broadcasted_iota(jnp.int32, (LN,), 0)


def _splat(x):
    return jnp.full((LN,), x, jnp.int32)


def _tables_kernel(f_ref, m_ref, ms_ref, fall_ref, sall_ref):
    f = f_ref[...]
    for v in range(4):
        fall_ref[v] = jax.lax.dot_general(
            f, m_ref[v], (((1,), (0,)), ((), ())),
            preferred_element_type=jnp.float32)
        sall_ref[v] = jax.lax.dot_general(
            f, ms_ref[v], (((1,), (0,)), ((), ())),
            preferred_element_type=jnp.float32)


def _build_tables(features, m, ms):
    blk = 1000
    grid = N_NODES // blk
    return pl.pallas_call(
        _tables_kernel,
        grid=(grid,),
        in_specs=[
            pl.BlockSpec((blk, IN_DIM), lambda i: (i, 0)),
            pl.BlockSpec((4, IN_DIM, IN_DIM), lambda i: (0, 0, 0)),
            pl.BlockSpec((4, IN_DIM, IN_DIM), lambda i: (0, 0, 0)),
        ],
        out_specs=[
            pl.BlockSpec((4, blk, IN_DIM), lambda i: (0, i, 0)),
            pl.BlockSpec((4, blk, IN_DIM), lambda i: (0, i, 0)),
        ],
        out_shape=[
            jax.ShapeDtypeStruct((4, N_NODES, IN_DIM), jnp.float32),
            jax.ShapeDtypeStruct((4, N_NODES, IN_DIM), jnp.float32),
        ],
    )(features, m, ms)


def _pass1_body(off0, off1, off2, h_off,
                pk_hbm, s_hbm,
                rec_hbm, asum_hbm,
                rowb, pkb, fidx, srows, recs, asuml, sem):
    wid = lax.axis_index("s") * 2 + lax.axis_index("c")
    iota = _iota16()
    zf = jnp.zeros((LN,), jnp.float32)
    offs = (off0, off1, off2)
    nrow = C1 // 8

    def init_asum(k, _):
        plsc.store_scatter(asuml, [iota + k * LN], zf)
        return 0
    lax.fori_loop(0, N_NODES * NUM_HEADS // LN, init_asum, 0)

    def chunk(c, _):
        e0 = wid * EPW + c * C1
        mk5 = iota < nrow
        plsc.store_scatter(rowb, [iota], (e0 // 8) + iota, mask=mk5)
        pltpu.async_copy(pk_hbm.at[rowb], pkb, sem).wait()

        def build(g, _):
            eloc = g * LN + iota
            mk = eloc < C1
            row = lax.shift_right_logical(eloc, 3)
            col0 = lax.bitwise_and(eloc, 7) * 16
            dstv = plsc.load_gather(pkb, [row, col0 + 3], mask=mk)
            plsc.store_scatter(recs, [iota * 16 + (g * 256 + 3)], dstv,
                               mask=mk)
            for t in range(L):
                it = plsc.load_gather(pkb, [row, col0 + t], mask=mk)
                ft = it + offs[t]
                plsc.store_scatter(fidx, [iota * 3 + (g * 48 + t)], ft,
                                   mask=mk)
                plsc.store_scatter(recs, [iota * 16 + (g * 256 + t)],
                                   ft, mask=mk)
            return 0
        lax.fori_loop(0, G1, build, 0)

        pltpu.async_copy(s_hbm.at[fidx], srows, sem).wait()

        def heads(g, _):
            mk = (g * LN + iota) < C1
            dstv = plsc.load_gather(recs, [iota * 16 + (g * 256 + 3)],
                                    mask=mk)
            for h in range(NUM_HEADS):
                col = h_off + h
                a = zf
                for t in range(L):
                    a = a + plsc.load_gather(
                        srows, [iota * 3 + (g * 48 + t), _splat(col)],
                        mask=mk)
                a = a * (1.0 / 3.0)
                a = jnp.maximum(a, 0.01 * a)
                ae = jnp.exp(a)
                plsc.addupdate_scatter(asuml, [dstv * NUM_HEADS + h], ae,
                                       mask=mk)
                plsc.store_scatter(recs,
                                   [iota * 16 + (g * 256 + 4 + h)],
                                   plsc.bitcast(ae, jnp.int32), mask=mk)
            return 0
        lax.fori_loop(0, G1, heads, 0)

        pltpu.sync_copy(recs, rec_hbm.at[pl.ds(e0 * 16, C1 * 16)])
        return 0
    lax.fori_loop(0, NCH1, chunk, 0)

    pltpu.sync_copy(asuml,
                    asum_hbm.at[pl.ds(wid * N_NODES * NUM_HEADS,
                                      N_NODES * NUM_HEADS)])


def _make_pass1(off0, off1, off2, h_off):
    mesh = plsc.VectorSubcoreMesh(core_axis_name="c", subcore_axis_name="s")
    return pl.kernel(
        functools.partial(_pass1_body, off0, off1, off2, h_off),
        out_type=[
            jax.ShapeDtypeStruct((E_MP * 16,), jnp.int32),
            jax.ShapeDtypeStruct((NW * N_NODES * NUM_HEADS,), jnp.float32),
        ],
        mesh=mesh,
        compiler_params=pltpu.CompilerParams(needs_layout_passes=False),
        scratch_types=[
            pltpu.VMEM((C1 // 8,), jnp.int32),
            pltpu.VMEM((C1 // 8, 128), jnp.int32),
            pltpu.VMEM((C1 * L,), jnp.int32),
            pltpu.VMEM((C1 * L, IN_DIM), jnp.float32),
            pltpu.VMEM((C1 * 16,), jnp.int32),
            pltpu.VMEM((N_NODES * NUM_HEADS,), jnp.float32),
            pltpu.SemaphoreType.DMA,
        ],
    )


def _pass2_body(rec_hbm, dst_hbm, asum_hbm, f_hbm, out_hbm,
                dstbuf, hits, rowbuf, recb, fidx2, frows, hidt, asumo, acc,
                sem):
    wid = lax.axis_index("s") * 2 + lax.axis_index("c")
    iota = _iota16()
    zf = jnp.zeros((LN,), jnp.float32)
    zi = jnp.zeros((LN,), jnp.int32)

    for k in range(SC_C // LN):
        plsc.store_scatter(hits, [iota + k * LN], zi)

    def round_body(r, _):
        nbase = (r * NW + wid) * NN

        def init_acc(k, _):
            plsc.store_scatter(acc, [iota + k * LN], zf)
            return 0
        lax.fori_loop(0, ACC_W // LN, init_acc, 0)

        blk = r * NW + wid
        pltpu.sync_copy(asum_hbm.at[pl.ds(blk * NN * NUM_HEADS,
                                          NN * NUM_HEADS)], asumo)

        def chunk(c, _):
            e0 = c * SC_C
            pltpu.sync_copy(dst_hbm.at[pl.ds(e0, SC_C)], dstbuf)

            def scan(g, nh):
                dv = plsc.load_gather(dstbuf, [iota + g * LN])
                mk = (dv >= nbase) & (dv < nbase + NN)
                mi = mk.astype(jnp.int32)
                pos = jnp.maximum(nh + plsc.cumsum(mi) - 1, 0)
                eid = e0 + g * LN + iota
                plsc.store_scatter(hits, [pos], eid, mask=mk)
                return nh + jnp.sum(mi)
            nh = lax.fori_loop(0, SC_C // LN, scan, jnp.int32(0))

            def sub(j, _):
                j0 = j * SUBC
                for g3 in range(SUBC // LN):
                    ej = plsc.load_gather(hits, [j0 + g3 * LN + iota])
                    plsc.store_scatter(rowbuf, [g3 * LN + iota],
                                       lax.shift_right_logical(ej, 3))
                pltpu.async_copy(rec_hbm.at[rowbuf], recb, sem).wait()
                for g3 in range(SUBC // LN):
                    ej = plsc.load_gather(hits, [j0 + g3 * LN + iota])
                    sub16 = lax.bitwise_and(ej, 7) * 16
                    for t in range(L):
                        ft = plsc.load_gather(
                            recb, [g3 * LN + iota, sub16 + t])
                        plsc.store_scatter(
                            fidx2, [iota * 3 + (g3 * 48 + t)], ft)
                pltpu.async_copy(f_hbm.at[fidx2], frows, sem).wait()

                def comb(e, _):
                    for dc in range(IN_DIM // LN):
                        v = (frows[e * 3, pl.ds(dc * LN, LN)]
                             + frows[e * 3 + 1, pl.ds(dc * LN, LN)]
                             + frows[e * 3 + 2, pl.ds(dc * LN, LN)])
                        plsc.store_scatter(
                            hidt, [(dc * LN + iota) * 97 + e],
                            v * (1.0 / 3.0))
                    return 0
                lax.fori_loop(0, SUBC, comb, 0)

                for g3 in range(SUBC // LN):
                    lanepos = j0 + g3 * LN + iota
                    mk2 = lanepos < nh
                    ej = plsc.load_gather(hits, [j0 + g3 * LN + iota])
                    sub16 = lax.bitwise_and(ej, 7) * 16
                    dv2 = plsc.load_gather(recb, [g3 * LN + iota, sub16 + 3])
                    dstl = jnp.clip(dv2 - nbase, 0, NN - 1)
                    ws = []
                    for h in range(NUM_HEADS):
                        aeh = plsc.bitcast(
                            plsc.load_gather(
                                recb, [g3 * LN + iota, sub16 + 4 + h]),
                            jnp.float32)
                        ash = plsc.load_gather(asumo, [h * NN + dstl])
                        ws.append(aeh / (ash + 1e-16))

                    def dloop(d, _):
                        hd = plsc.load_gather(
                            hidt, [iota + (g3 * LN + d * 97)])
                        for h in range(NUM_HEADS):
                            plsc.addupdate_scatter(
                                acc, [dstl + (h * IN_DIM + d) * NN],
                                ws[h] * hd, mask=mk2)
                        return 0
                    lax.fori_loop(0, IN_DIM, dloop, 0)
                return 0
            lax.fori_loop(0, (nh + SUBC - 1) // SUBC, sub, 0)
            return 0
        lax.fori_loop(0, NSCCH, chunk, 0)

        pltpu.sync_copy(acc, out_hbm.at[pl.ds(nbase * NUM_HEADS * IN_DIM,
                                              ACC_W)])
        return 0
    lax.fori_loop(0, ROUNDS, round_body, 0)


def _make_pass2():
    mesh = plsc.VectorSubcoreMesh(core_axis_name="c", subcore_axis_name="s")
    return pl.kernel(
        _pass2_body,
        out_type=jax.ShapeDtypeStruct((NPAD * NUM_HEADS * IN_DIM,),
                                      jnp.float32),
        mesh=mesh,
        compiler_params=pltpu.CompilerParams(needs_layout_passes=False),
        scratch_types=[
            pltpu.VMEM((SC_C,), jnp.int32),
            pltpu.VMEM((SC_C,), jnp.int32),
            pltpu.VMEM((SUBC,), jnp.int32),
            pltpu.VMEM((SUBC, 128), jnp.int32),
            pltpu.VMEM((SUBC * L,), jnp.int32),
            pltpu.VMEM((SUBC * L, IN_DIM), jnp.float32),
            pltpu.VMEM((12336,), jnp.float32),
            pltpu.VMEM((NN * NUM_HEADS,), jnp.float32),
            pltpu.VMEM((ACC_W,), jnp.float32),
            pltpu.SemaphoreType.DMA,
        ],
    )


def _b1_kernel(ft0_ref, ft1_ref, w_ref, b_ref, o0_ref, o1_ref, p_ref):
    for i, (ft_ref, o_ref) in enumerate([(ft0_ref, o0_ref),
                                         (ft1_ref, o1_ref)]):
        x = ft_ref[...]
        o = jnp.where(x > 0, x, jnp.exp(jnp.minimum(x, 0.0)) - 1.0)
        o_ref[...] = o
        f1 = jnp.tanh(jax.lax.dot_general(
            o, w_ref[...], (((1,), (1,)), ((), ())),
            preferred_element_type=jnp.float32) + b_ref[...][None, :])
        p_ref[0, i] = jnp.sum(f1, axis=0)


def _run_b1(ft0, ft1, fc1_w, fc1_b):
    blk = 1000
    grid = N_NODES // blk
    dh = NUM_HEADS * IN_DIM
    return pl.pallas_call(
        _b1_kernel,
        grid=(grid,),
        in_specs=[
            pl.BlockSpec((blk, dh), lambda i: (i, 0)),
            pl.BlockSpec((blk, dh), lambda i: (i, 0)),
            pl.BlockSpec((128, dh), lambda i: (0, 0)),
            pl.BlockSpec((128,), lambda i: (0,)),
        ],
        out_specs=[
            pl.BlockSpec((blk, dh), lambda i: (i, 0)),
            pl.BlockSpec((blk, dh), lambda i: (i, 0)),
            pl.BlockSpec((1, 2, 128), lambda i: (i, 0, 0)),
        ],
        out_shape=[
            jax.ShapeDtypeStruct((N_NODES, dh), jnp.float32),
            jax.ShapeDtypeStruct((N_NODES, dh), jnp.float32),
            jax.ShapeDtypeStruct((grid, 2, 128), jnp.float32),
        ],
    )(ft0, ft1, fc1_w, fc1_b)


def _final_kernel(o0_ref, o1_ref, beta_ref, fcw_ref, fcb_ref, h_ref, hfc_ref):
    b0 = beta_ref[0, 0]
    b1 = beta_ref[0, 1]
    h = b0 * o0_ref[...] + b1 * o1_ref[...]
    h_ref[...] = h
    hfc_ref[...] = jax.lax.dot_general(
        h, fcw_ref[...], (((1,), (1,)), ((), ())),
        preferred_element_type=jnp.float32) + fcb_ref[...][None, :]


def _run_final(o0, o1, beta, fc_w, fc_b):
    blk = 1000
    grid = N_NODES // blk
    dh = NUM_HEADS * IN_DIM
    h, h_fc = pl.pallas_call(
        _final_kernel,
        grid=(grid,),
        in_specs=[
            pl.BlockSpec((blk, dh), lambda i: (i, 0)),
            pl.BlockSpec((blk, dh), lambda i: (i, 0)),
            pl.BlockSpec((1, 2), lambda i: (0, 0)),
            pl.BlockSpec((OUT_DIM, dh), lambda i: (0, 0)),
            pl.BlockSpec((OUT_DIM,), lambda i: (0,)),
        ],
        out_specs=[
            pl.BlockSpec((blk, dh), lambda i: (i, 0)),
            pl.BlockSpec((blk, OUT_DIM), lambda i: (i, 0)),
        ],
        out_shape=[
            jax.ShapeDtypeStruct((N_NODES, dh), jnp.float32),
            jax.ShapeDtypeStruct((N_NODES, OUT_DIM), jnp.float32),
        ],
    )(o0, o1, beta.reshape(1, 2), fc_w, fc_b)
    return h, h_fc


def _rot_matrices(r_vec):
    rv = r_vec / (jnp.linalg.norm(r_vec, axis=2, keepdims=True) + 1e-12)
    r0re, r0im = rv[0, :, 0], rv[0, :, 1]
    r1re, r1im = rv[1, :, 0], rv[1, :, 1]
    p_re = r0re * r1re - r0im * r1im
    p_im = r0re * r1im + r0im * r1re
    ident = (jnp.ones(IN_DIM // 2), jnp.zeros(IN_DIM // 2))
    eye = jnp.eye(IN_DIM // 2, dtype=jnp.float32)
    mats = []
    for re, im in [(p_re, p_im), (r1re, r1im), (r0re, r0im), ident]:
        r2 = jnp.stack([jnp.stack([re, im]), jnp.stack([-im, re])])
        mats.append(jnp.einsum('kl,abk->kalb', eye, r2).reshape(IN_DIM,
                                                                IN_DIM))
    return jnp.stack(mats)


def kernel(features, type_mask, edge_metapath_indices_0,
           edge_metapath_indices_1, dst_0, dst_1, r_vec, attn_0, attn_1,
           fc1_w, fc1_b, fc2_w, fc_w, fc_b):
    attn_cat = jnp.concatenate([attn_0, attn_1], axis=0)
    m = _rot_matrices(r_vec)
    ms_pad = jnp.pad(jnp.einsum('vij,hj->vih', m, attn_cat),
                     ((0, 0), (0, 0), (0, IN_DIM - 16)))
    f_all, s_all = _build_tables(features, m, ms_pad)
    f_flat = f_all.reshape(4 * N_NODES, IN_DIM)
    s_flat = s_all.reshape(4 * N_NODES, IN_DIM)

    pass1s = [_make_pass1(0, N_NODES, 3 * N_NODES, 0),
              _make_pass1(0, 2 * N_NODES, 3 * N_NODES, 8)]
    pass2 = _make_pass2()
    fts = []
    for mp, (idx, dst) in enumerate([(edge_metapath_indices_0, dst_0),
                                     (edge_metapath_indices_1, dst_1)]):
        idx32 = idx.astype(jnp.int32)
        dst32 = dst.astype(jnp.int32)
        pk = jnp.concatenate(
            [idx32, dst32[:, None],
             jnp.zeros((E_MP, 12), jnp.int32)], axis=1).reshape(RECR, 128)
        rec, parts = pass1s[mp](pk, s_flat)
        asum = parts.reshape(NW, N_NODES * NUM_HEADS).sum(axis=0)
        asum_pad = jnp.pad(asum, (0, (NPAD - N_NODES) * NUM_HEADS))
        asum_blk = asum_pad.reshape(NW * ROUNDS, NN, NUM_HEADS).transpose(
            0, 2, 1).reshape(-1)
        ftflat = pass2(rec.reshape(RECR, 128), dst32, asum_blk, f_flat)
        ft = ftflat.reshape(NW * ROUNDS, NUM_HEADS * IN_DIM, NN).transpose(
            0, 2, 1).reshape(NPAD, NUM_HEADS * IN_DIM)
        fts.append(ft[:N_NODES])

    o0, o1, p = _run_b1(fts[0], fts[1], fc1_w, fc1_b)
    f1m = p.sum(axis=0) * (1.0 / N_NODES)
    betas = f1m @ fc2_w[0]
    beta = jax.nn.softmax(betas)
    h, h_fc = _run_final(o0, o1, beta, fc_w, fc_b)
    return (h_fc, h)


# speedup vs baseline: 15.3825x; 1.0653x over previous
import functools
import jax
import jax.numpy as jnp
from jax import lax
from jax.experimental import pallas as pl
from jax.experimental.pallas import tpu as pltpu
from jax.experimental.pallas import tpu_sc as plsc

N_NODES = 10000
IN_DIM = 128
OUT_DIM = 128
NUM_HEADS = 8
E_MP = 160000
L = 3

NW = 32
LN = 16
EPW = E_MP // NW
C1 = 40
NCH1 = EPW // C1
G1 = 3
RECR = E_MP // 8

NN = 96
ROUNDS = 4
NPAD = NN * NW * ROUNDS
SC_C = 3200
NSCCH = E_MP // SC_C
SUBC = 16
ACC_W = NN * NUM_HEADS * IN_DIM


def _iota16():
    return lax.broadcasted_iota(jnp.int32, (LN,), 0)


def _splat(x):
    return jnp.full((LN,), x, jnp.int32)


def _tables_kernel(f_ref, m_ref, ms_ref, fall_ref, sall_ref):
    f = f_ref[...]
    for v in range(4):
        fall_ref[v] = jax.lax.dot_general(
            f, m_ref[v], (((1,), (0,)), ((), ())),
            preferred_element_type=jnp.float32)
        sall_ref[v] = jax.lax.dot_general(
            f, ms_ref[v], (((1,), (0,)), ((), ())),
            preferred_element_type=jnp.float32)


def _build_tables(features, m, ms):
    blk = 1000
    grid = N_NODES // blk
    return pl.pallas_call(
        _tables_kernel,
        grid=(grid,),
        in_specs=[
            pl.BlockSpec((blk, IN_DIM), lambda i: (i, 0)),
            pl.BlockSpec((4, IN_DIM, IN_DIM), lambda i: (0, 0, 0)),
            pl.BlockSpec((4, IN_DIM, IN_DIM), lambda i: (0, 0, 0)),
        ],
        out_specs=[
            pl.BlockSpec((4, blk, IN_DIM), lambda i: (0, i, 0)),
            pl.BlockSpec((4, blk, IN_DIM), lambda i: (0, i, 0)),
        ],
        out_shape=[
            jax.ShapeDtypeStruct((4, N_NODES, IN_DIM), jnp.float32),
            jax.ShapeDtypeStruct((4, N_NODES, IN_DIM), jnp.float32),
        ],
    )(features, m, ms)


def _pass1_body(off0, off1, off2, h_off,
                pk_hbm, s_hbm,
                rec_hbm, asum_hbm,
                rowb, pkb, fidx, srows, recs, asuml, sem):
    wid = lax.axis_index("s") * 2 + lax.axis_index("c")
    iota = _iota16()
    zf = jnp.zeros((LN,), jnp.float32)
    offs = (off0, off1, off2)
    nrow = C1 // 8

    def init_asum(k, _):
        plsc.store_scatter(asuml, [iota + k * LN], zf)
        return 0
    lax.fori_loop(0, N_NODES * NUM_HEADS // LN, init_asum, 0)

    def chunk(c, _):
        e0 = wid * EPW + c * C1
        mk5 = iota < nrow
        plsc.store_scatter(rowb, [iota], (e0 // 8) + iota, mask=mk5)
        pltpu.async_copy(pk_hbm.at[rowb], pkb, sem).wait()

        def build(g, _):
            eloc = g * LN + iota
            mk = eloc < C1
            row = lax.shift_right_logical(eloc, 3)
            col0 = lax.bitwise_and(eloc, 7) * 16
            dstv = plsc.load_gather(pkb, [row, col0 + 3], mask=mk)
            plsc.store_scatter(recs, [iota * 16 + (g * 256 + 3)], dstv,
                               mask=mk)
            for t in range(L):
                it = plsc.load_gather(pkb, [row, col0 + t], mask=mk)
                ft = it + offs[t]
                plsc.store_scatter(fidx, [iota * 3 + (g * 48 + t)], ft,
                                   mask=mk)
                plsc.store_scatter(recs, [iota * 16 + (g * 256 + t)],
                                   ft, mask=mk)
            return 0
        lax.fori_loop(0, G1, build, 0)

        pltpu.async_copy(s_hbm.at[fidx], srows, sem).wait()

        def heads(g, _):
            mk = (g * LN + iota) < C1
            dstv = plsc.load_gather(recs, [iota * 16 + (g * 256 + 3)],
                                    mask=mk)
            for h in range(NUM_HEADS):
                col = h_off + h
                a = zf
                for t in range(L):
                    a = a + plsc.load_gather(
                        srows, [iota * 3 + (g * 48 + t), _splat(col)],
                        mask=mk)
                a = a * (1.0 / 3.0)
                a = jnp.maximum(a, 0.01 * a)
                ae = jnp.exp(a)
                plsc.addupdate_scatter(asuml, [dstv * NUM_HEADS + h], ae,
                                       mask=mk)
                plsc.store_scatter(recs,
                                   [iota * 16 + (g * 256 + 4 + h)],
                                   plsc.bitcast(ae, jnp.int32), mask=mk)
            return 0
        lax.fori_loop(0, G1, heads, 0)

        pltpu.sync_copy(recs, rec_hbm.at[pl.ds(e0 * 16, C1 * 16)])
        return 0
    lax.fori_loop(0, NCH1, chunk, 0)

    pltpu.sync_copy(asuml,
                    asum_hbm.at[pl.ds(wid * N_NODES * NUM_HEADS,
                                      N_NODES * NUM_HEADS)])


def _make_pass1(off0, off1, off2, h_off):
    mesh = plsc.VectorSubcoreMesh(core_axis_name="c", subcore_axis_name="s")
    return pl.kernel(
        functools.partial(_pass1_body, off0, off1, off2, h_off),
        out_type=[
            jax.ShapeDtypeStruct((E_MP * 16,), jnp.int32),
            jax.ShapeDtypeStruct((NW * N_NODES * NUM_HEADS,), jnp.float32),
        ],
        mesh=mesh,
        compiler_params=pltpu.CompilerParams(needs_layout_passes=False),
        scratch_types=[
            pltpu.VMEM((C1 // 8,), jnp.int32),
            pltpu.VMEM((C1 // 8, 128), jnp.int32),
            pltpu.VMEM((C1 * L,), jnp.int32),
            pltpu.VMEM((C1 * L, IN_DIM), jnp.float32),
            pltpu.VMEM((C1 * 16,), jnp.int32),
            pltpu.VMEM((N_NODES * NUM_HEADS,), jnp.float32),
            pltpu.SemaphoreType.DMA,
        ],
    )


def _pass2_body(rec_hbm, dst_hbm, asum_hbm, f_hbm, out_hbm,
                dstbuf, hits, rowbuf, recb, fidx2, frows, hidt, asumo, acc,
                sem):
    wid = lax.axis_index("s") * 2 + lax.axis_index("c")
    iota = _iota16()
    zf = jnp.zeros((LN,), jnp.float32)
    zi = jnp.zeros((LN,), jnp.int32)

    for k in range(SC_C // LN):
        plsc.store_scatter(hits, [iota + k * LN], zi)

    def round_body(r, _):
        nbase = (r * NW + wid) * NN

        def init_acc(k, _):
            plsc.store_scatter(acc, [iota + k * LN], zf)
            return 0
        lax.fori_loop(0, ACC_W // LN, init_acc, 0)

        blk = r * NW + wid
        pltpu.sync_copy(asum_hbm.at[pl.ds(blk * NN * NUM_HEADS,
                                          NN * NUM_HEADS)], asumo)

        def chunk(c, _):
            e0 = c * SC_C
            pltpu.sync_copy(dst_hbm.at[pl.ds(e0, SC_C)], dstbuf)

            lane15 = _splat(15)

            def scan2(gg, nhv):
                g0 = gg * 2
                dv0 = plsc.load_gather(dstbuf, [iota + g0 * LN])
                dv1 = plsc.load_gather(dstbuf, [iota + (g0 + 1) * LN])
                mk0 = (dv0 >= nbase) & (dv0 < nbase + NN)
                mk1 = (dv1 >= nbase) & (dv1 < nbase + NN)
                cs0 = plsc.cumsum(mk0.astype(jnp.int32))
                cs1 = plsc.cumsum(mk1.astype(jnp.int32))
                bc0 = cs0.at[lane15].get(mode="promise_in_bounds")
                pos0 = jnp.maximum(nhv + cs0 - 1, 0)
                pos1 = jnp.maximum(nhv + bc0 + cs1 - 1, 0)
                plsc.store_scatter(hits, [pos0], e0 + g0 * LN + iota,
                                   mask=mk0)
                plsc.store_scatter(hits, [pos1], e0 + (g0 + 1) * LN + iota,
                                   mask=mk1)
                return nhv + bc0 + cs1.at[lane15].get(mode="promise_in_bounds")
            nhv = lax.fori_loop(0, SC_C // (2 * LN), scan2,
                                jnp.zeros((LN,), jnp.int32))
            nh = jnp.max(nhv)

            def sub(j, _):
                j0 = j * SUBC
                for g3 in range(SUBC // LN):
                    ej = plsc.load_gather(hits, [j0 + g3 * LN + iota])
                    plsc.store_scatter(rowbuf, [g3 * LN + iota],
                                       lax.shift_right_logical(ej, 3))
                pltpu.async_copy(rec_hbm.at[rowbuf], recb, sem).wait()
                for g3 in range(SUBC // LN):
                    ej = plsc.load_gather(hits, [j0 + g3 * LN + iota])
                    sub16 = lax.bitwise_and(ej, 7) * 16
                    for t in range(L):
                        ft = plsc.load_gather(
                            recb, [g3 * LN + iota, sub16 + t])
                        plsc.store_scatter(
                            fidx2, [iota * 3 + (g3 * 48 + t)], ft)
                pltpu.async_copy(f_hbm.at[fidx2], frows, sem).wait()

                def comb(e, _):
                    for dc in range(IN_DIM // LN):
                        v = (frows[e * 3, pl.ds(dc * LN, LN)]
                             + frows[e * 3 + 1, pl.ds(dc * LN, LN)]
                             + frows[e * 3 + 2, pl.ds(dc * LN, LN)])
                        plsc.store_scatter(
                            hidt, [(dc * LN + iota) * 97 + e],
                            v * (1.0 / 3.0))
                    return 0
                lax.fori_loop(0, SUBC, comb, 0)

                for g3 in range(SUBC // LN):
                    lanepos = j0 + g3 * LN + iota
                    mk2 = lanepos < nh
                    ej = plsc.load_gather(hits, [j0 + g3 * LN + iota])
                    sub16 = lax.bitwise_and(ej, 7) * 16
                    dv2 = plsc.load_gather(recb, [g3 * LN + iota, sub16 + 3])
                    dstl = jnp.clip(dv2 - nbase, 0, NN - 1)
                    ws = []
                    for h in range(NUM_HEADS):
                        aeh = plsc.bitcast(
                            plsc.load_gather(
                                recb, [g3 * LN + iota, sub16 + 4 + h]),
                            jnp.float32)
                        ash = plsc.load_gather(asumo, [h * NN + dstl])
                        ws.append(aeh / (ash + 1e-16))

                    def dloop(d, _):
                        hd = plsc.load_gather(
                            hidt, [iota + (g3 * LN + d * 97)])
                        for h in range(NUM_HEADS):
                            plsc.addupdate_scatter(
                                acc, [dstl + (h * IN_DIM + d) * NN],
                                ws[h] * hd, mask=mk2)
                        return 0
                    lax.fori_loop(0, IN_DIM, dloop, 0)
                return 0
            lax.fori_loop(0, (nh + SUBC - 1) // SUBC, sub, 0)
            return 0
        lax.fori_loop(0, NSCCH, chunk, 0)

        pltpu.sync_copy(acc, out_hbm.at[pl.ds(nbase * NUM_HEADS * IN_DIM,
                                              ACC_W)])
        return 0
    lax.fori_loop(0, ROUNDS, round_body, 0)


def _make_pass2():
    mesh = plsc.VectorSubcoreMesh(core_axis_name="c", subcore_axis_name="s")
    return pl.kernel(
        _pass2_body,
        out_type=jax.ShapeDtypeStruct((NPAD * NUM_HEADS * IN_DIM,),
                                      jnp.float32),
        mesh=mesh,
        compiler_params=pltpu.CompilerParams(needs_layout_passes=False),
        scratch_types=[
            pltpu.VMEM((SC_C,), jnp.int32),
            pltpu.VMEM((SC_C,), jnp.int32),
            pltpu.VMEM((SUBC,), jnp.int32),
            pltpu.VMEM((SUBC, 128), jnp.int32),
            pltpu.VMEM((SUBC * L,), jnp.int32),
            pltpu.VMEM((SUBC * L, IN_DIM), jnp.float32),
            pltpu.VMEM((12336,), jnp.float32),
            pltpu.VMEM((NN * NUM_HEADS,), jnp.float32),
            pltpu.VMEM((ACC_W,), jnp.float32),
            pltpu.SemaphoreType.DMA,
        ],
    )


def _b1_kernel(ft0_ref, ft1_ref, w_ref, b_ref, o0_ref, o1_ref, p_ref):
    for i, (ft_ref, o_ref) in enumerate([(ft0_ref, o0_ref),
                                         (ft1_ref, o1_ref)]):
        x = ft_ref[...]
        o = jnp.where(x > 0, x, jnp.exp(jnp.minimum(x, 0.0)) - 1.0)
        o_ref[...] = o
        f1 = jnp.tanh(jax.lax.dot_general(
            o, w_ref[...], (((1,), (1,)), ((), ())),
            preferred_element_type=jnp.float32) + b_ref[...][None, :])
        p_ref[0, i] = jnp.sum(f1, axis=0)


def _run_b1(ft0, ft1, fc1_w, fc1_b):
    blk = 1000
    grid = N_NODES // blk
    dh = NUM_HEADS * IN_DIM
    return pl.pallas_call(
        _b1_kernel,
        grid=(grid,),
        in_specs=[
            pl.BlockSpec((blk, dh), lambda i: (i, 0)),
            pl.BlockSpec((blk, dh), lambda i: (i, 0)),
            pl.BlockSpec((128, dh), lambda i: (0, 0)),
            pl.BlockSpec((128,), lambda i: (0,)),
        ],
        out_specs=[
            pl.BlockSpec((blk, dh), lambda i: (i, 0)),
            pl.BlockSpec((blk, dh), lambda i: (i, 0)),
            pl.BlockSpec((1, 2, 128), lambda i: (i, 0, 0)),
        ],
        out_shape=[
            jax.ShapeDtypeStruct((N_NODES, dh), jnp.float32),
            jax.ShapeDtypeStruct((N_NODES, dh), jnp.float32),
            jax.ShapeDtypeStruct((grid, 2, 128), jnp.float32),
        ],
    )(ft0, ft1, fc1_w, fc1_b)


def _final_kernel(o0_ref, o1_ref, beta_ref, fcw_ref, fcb_ref, h_ref, hfc_ref):
    b0 = beta_ref[0, 0]
    b1 = beta_ref[0, 1]
    h = b0 * o0_ref[...] + b1 * o1_ref[...]
    h_ref[...] = h
    hfc_ref[...] = jax.lax.dot_general(
        h, fcw_ref[...], (((1,), (1,)), ((), ())),
        preferred_element_type=jnp.float32) + fcb_ref[...][None, :]


def _run_final(o0, o1, beta, fc_w, fc_b):
    blk = 1000
    grid = N_NODES // blk
    dh = NUM_HEADS * IN_DIM
    h, h_fc = pl.pallas_call(
        _final_kernel,
        grid=(grid,),
        in_specs=[
            pl.BlockSpec((blk, dh), lambda i: (i, 0)),
            pl.BlockSpec((blk, dh), lambda i: (i, 0)),
            pl.BlockSpec((1, 2), lambda i: (0, 0)),
            pl.BlockSpec((OUT_DIM, dh), lambda i: (0, 0)),
            pl.BlockSpec((OUT_DIM,), lambda i: (0,)),
        ],
        out_specs=[
            pl.BlockSpec((blk, dh), lambda i: (i, 0)),
            pl.BlockSpec((blk, OUT_DIM), lambda i: (i, 0)),
        ],
        out_shape=[
            jax.ShapeDtypeStruct((N_NODES, dh), jnp.float32),
            jax.ShapeDtypeStruct((N_NODES, OUT_DIM), jnp.float32),
        ],
    )(o0, o1, beta.reshape(1, 2), fc_w, fc_b)
    return h, h_fc


def _rot_matrices(r_vec):
    rv = r_vec / (jnp.linalg.norm(r_vec, axis=2, keepdims=True) + 1e-12)
    r0re, r0im = rv[0, :, 0], rv[0, :, 1]
    r1re, r1im = rv[1, :, 0], rv[1, :, 1]
    p_re = r0re * r1re - r0im * r1im
    p_im = r0re * r1im + r0im * r1re
    ident = (jnp.ones(IN_DIM // 2), jnp.zeros(IN_DIM // 2))
    eye = jnp.eye(IN_DIM // 2, dtype=jnp.float32)
    mats = []
    for re, im in [(p_re, p_im), (r1re, r1im), (r0re, r0im), ident]:
        r2 = jnp.stack([jnp.stack([re, im]), jnp.stack([-im, re])])
        mats.append(jnp.einsum('kl,abk->kalb', eye, r2).reshape(IN_DIM,
                                                                IN_DIM))
    return jnp.stack(mats)


def kernel(features, type_mask, edge_metapath_indices_0,
           edge_metapath_indices_1, dst_0, dst_1, r_vec, attn_0, attn_1,
           fc1_w, fc1_b, fc2_w, fc_w, fc_b):
    attn_cat = jnp.concatenate([attn_0, attn_1], axis=0)
    m = _rot_matrices(r_vec)
    ms_pad = jnp.pad(jnp.einsum('vij,hj->vih', m, attn_cat),
                     ((0, 0), (0, 0), (0, IN_DIM - 16)))
    f_all, s_all = _build_tables(features, m, ms_pad)
    f_flat = f_all.reshape(4 * N_NODES, IN_DIM)
    s_flat = s_all.reshape(4 * N_NODES, IN_DIM)

    pass1s = [_make_pass1(0, N_NODES, 3 * N_NODES, 0),
              _make_pass1(0, 2 * N_NODES, 3 * N_NODES, 8)]
    pass2 = _make_pass2()
    fts = []
    for mp, (idx, dst) in enumerate([(edge_metapath_indices_0, dst_0),
                                     (edge_metapath_indices_1, dst_1)]):
        idx32 = idx.astype(jnp.int32)
        dst32 = dst.astype(jnp.int32)
        pk = jnp.concatenate(
            [idx32, dst32[:, None],
             jnp.zeros((E_MP, 12), jnp.int32)], axis=1).reshape(RECR, 128)
        rec, parts = pass1s[mp](pk, s_flat)
        asum = parts.reshape(NW, N_NODES * NUM_HEADS).sum(axis=0)
        asum_pad = jnp.pad(asum, (0, (NPAD - N_NODES) * NUM_HEADS))
        asum_blk = asum_pad.reshape(NW * ROUNDS, NN, NUM_HEADS).transpose(
            0, 2, 1).reshape(-1)
        ftflat = pass2(rec.reshape(RECR, 128), dst32, asum_blk, f_flat)
        ft = ftflat.reshape(NW * ROUNDS, NUM_HEADS * IN_DIM, NN).transpose(
            0, 2, 1).reshape(NPAD, NUM_HEADS * IN_DIM)
        fts.append(ft[:N_NODES])

    o0, o1, p = _run_b1(fts[0], fts[1], fc1_w, fc1_b)
    f1m = p.sum(axis=0) * (1.0 / N_NODES)
    betas = f1m @ fc2_w[0]
    beta = jax.nn.softmax(betas)
    h, h_fc = _run_final(o0, o1, beta, fc_w, fc_b)
    return (h_fc, h)


# speedup vs baseline: 15.3897x; 1.0005x over previous
import functools
import jax
import jax.numpy as jnp
from jax import lax
from jax.experimental import pallas as pl
from jax.experimental.pallas import tpu as pltpu
from jax.experimental.pallas import tpu_sc as plsc

N_NODES = 10000
IN_DIM = 128
OUT_DIM = 128
NUM_HEADS = 8
E_MP = 160000
L = 3

NW = 32
LN = 16
EPW = E_MP // NW
C1 = 40
NCH1 = EPW // C1
G1 = 3
RECR = E_MP // 8

NN = 96
ROUNDS = 4
NPAD = NN * NW * ROUNDS
SC_C = 3200
NSCCH = E_MP // SC_C
SUBC = 16
ACC_W = NN * NUM_HEADS * IN_DIM


def _iota16():
    return lax.broadcasted_iota(jnp.int32, (LN,), 0)


def _splat(x):
    return jnp.full((LN,), x, jnp.int32)


def _tables_kernel(f_ref, m_ref, ms_ref, fall_ref, sall_ref):
    f = f_ref[...]
    for v in range(4):
        fall_ref[v] = jax.lax.dot_general(
            f, m_ref[v], (((1,), (0,)), ((), ())),
            preferred_element_type=jnp.float32)
        sall_ref[v] = jax.lax.dot_general(
            f, ms_ref[v], (((1,), (0,)), ((), ())),
            preferred_element_type=jnp.float32)


def _build_tables(features, m, ms):
    blk = 1000
    grid = N_NODES // blk
    return pl.pallas_call(
        _tables_kernel,
        grid=(grid,),
        in_specs=[
            pl.BlockSpec((blk, IN_DIM), lambda i: (i, 0)),
            pl.BlockSpec((4, IN_DIM, IN_DIM), lambda i: (0, 0, 0)),
            pl.BlockSpec((4, IN_DIM, IN_DIM), lambda i: (0, 0, 0)),
        ],
        out_specs=[
            pl.BlockSpec((4, blk, IN_DIM), lambda i: (0, i, 0)),
            pl.BlockSpec((4, blk, IN_DIM), lambda i: (0, i, 0)),
        ],
        out_shape=[
            jax.ShapeDtypeStruct((4, N_NODES, IN_DIM), jnp.float32),
            jax.ShapeDtypeStruct((4, N_NODES, IN_DIM), jnp.float32),
        ],
    )(features, m, ms)


def _pass1_body(off0, off1, off2, h_off,
                pk_hbm, s_hbm,
                rec_hbm, asum_hbm,
                rowb, pkb, fidx, srows, recs, asuml, sem):
    wid = lax.axis_index("s") * 2 + lax.axis_index("c")
    iota = _iota16()
    zf = jnp.zeros((LN,), jnp.float32)
    offs = (off0, off1, off2)
    nrow = C1 // 8

    def init_asum(k, _):
        plsc.store_scatter(asuml, [iota + k * LN], zf)
        return 0
    lax.fori_loop(0, N_NODES * NUM_HEADS // LN, init_asum, 0)

    def chunk(c, _):
        e0 = wid * EPW + c * C1
        mk5 = iota < nrow
        plsc.store_scatter(rowb, [iota], (e0 // 8) + iota, mask=mk5)
        pltpu.async_copy(pk_hbm.at[rowb], pkb, sem).wait()

        def build(g, _):
            eloc = g * LN + iota
            mk = eloc < C1
            row = lax.shift_right_logical(eloc, 3)
            col0 = lax.bitwise_and(eloc, 7) * 16
            dstv = plsc.load_gather(pkb, [row, col0 + 3], mask=mk)
            plsc.store_scatter(recs, [iota * 16 + (g * 256 + 3)], dstv,
                               mask=mk)
            for t in range(L):
                it = plsc.load_gather(pkb, [row, col0 + t], mask=mk)
                ft = it + offs[t]
                plsc.store_scatter(fidx, [iota * 3 + (g * 48 + t)], ft,
                                   mask=mk)
                plsc.store_scatter(recs, [iota * 16 + (g * 256 + t)],
                                   ft, mask=mk)
            return 0
        lax.fori_loop(0, G1, build, 0)

        pltpu.async_copy(s_hbm.at[fidx], srows, sem).wait()

        def heads(g, _):
            mk = (g * LN + iota) < C1
            dstv = plsc.load_gather(recs, [iota * 16 + (g * 256 + 3)],
                                    mask=mk)
            for h in range(NUM_HEADS):
                col = h_off + h
                a = zf
                for t in range(L):
                    a = a + plsc.load_gather(
                        srows, [iota * 3 + (g * 48 + t), _splat(col)],
                        mask=mk)
                a = a * (1.0 / 3.0)
                a = jnp.maximum(a, 0.01 * a)
                ae = jnp.exp(a)
                plsc.addupdate_scatter(asuml, [dstv * NUM_HEADS + h], ae,
                                       mask=mk)
                plsc.store_scatter(recs,
                                   [iota * 16 + (g * 256 + 4 + h)],
                                   plsc.bitcast(ae, jnp.int32), mask=mk)
            return 0
        lax.fori_loop(0, G1, heads, 0)

        pltpu.sync_copy(recs, rec_hbm.at[pl.ds(e0 * 16, C1 * 16)])
        return 0
    lax.fori_loop(0, NCH1, chunk, 0)

    pltpu.sync_copy(asuml,
                    asum_hbm.at[pl.ds(wid * N_NODES * NUM_HEADS,
                                      N_NODES * NUM_HEADS)])


def _make_pass1(off0, off1, off2, h_off):
    mesh = plsc.VectorSubcoreMesh(core_axis_name="c", subcore_axis_name="s")
    return pl.kernel(
        functools.partial(_pass1_body, off0, off1, off2, h_off),
        out_type=[
            jax.ShapeDtypeStruct((E_MP * 16,), jnp.int32),
            jax.ShapeDtypeStruct((NW * N_NODES * NUM_HEADS,), jnp.float32),
        ],
        mesh=mesh,
        compiler_params=pltpu.CompilerParams(needs_layout_passes=False),
        scratch_types=[
            pltpu.VMEM((C1 // 8,), jnp.int32),
            pltpu.VMEM((C1 // 8, 128), jnp.int32),
            pltpu.VMEM((C1 * L,), jnp.int32),
            pltpu.VMEM((C1 * L, IN_DIM), jnp.float32),
            pltpu.VMEM((C1 * 16,), jnp.int32),
            pltpu.VMEM((N_NODES * NUM_HEADS,), jnp.float32),
            pltpu.SemaphoreType.DMA,
        ],
    )


def _pass2_body(rec_hbm, dst_hbm, asum_hbm, f_hbm, out_hbm,
                dstbuf, hits, rowbuf, recb, fidx2, frows, hidt, asumo, acc,
                sem):
    wid = lax.axis_index("s") * 2 + lax.axis_index("c")
    iota = _iota16()
    zf = jnp.zeros((LN,), jnp.float32)
    zi = jnp.zeros((LN,), jnp.int32)

    for k in range(SC_C // LN):
        plsc.store_scatter(hits, [iota + k * LN], zi)

    def round_body(r, _):
        nbase = (r * NW + wid) * NN
        return lax.cond(nbase < N_NODES,
                        lambda: _round_work(r, nbase),
                        lambda: 0)

    def _round_work(r, nbase):

        def init_acc(k, _):
            plsc.store_scatter(acc, [iota + k * LN], zf)
            return 0
        lax.fori_loop(0, ACC_W // LN, init_acc, 0)

        blk = r * NW + wid
        pltpu.sync_copy(asum_hbm.at[pl.ds(blk * NN * NUM_HEADS,
                                          NN * NUM_HEADS)], asumo)

        def chunk(c, _):
            e0 = c * SC_C
            pltpu.sync_copy(dst_hbm.at[pl.ds(e0, SC_C)], dstbuf)

            lane15 = _splat(15)

            def scan2(gg, nhv):
                g0 = gg * 2
                dv0 = plsc.load_gather(dstbuf, [iota + g0 * LN])
                dv1 = plsc.load_gather(dstbuf, [iota + (g0 + 1) * LN])
                mk0 = (dv0 >= nbase) & (dv0 < nbase + NN)
                mk1 = (dv1 >= nbase) & (dv1 < nbase + NN)
                cs0 = plsc.cumsum(mk0.astype(jnp.int32))
                cs1 = plsc.cumsum(mk1.astype(jnp.int32))
                bc0 = cs0.at[lane15].get(mode="promise_in_bounds")
                pos0 = jnp.maximum(nhv + cs0 - 1, 0)
                pos1 = jnp.maximum(nhv + bc0 + cs1 - 1, 0)
                plsc.store_scatter(hits, [pos0], e0 + g0 * LN + iota,
                                   mask=mk0)
                plsc.store_scatter(hits, [pos1], e0 + (g0 + 1) * LN + iota,
                                   mask=mk1)
                return nhv + bc0 + cs1.at[lane15].get(mode="promise_in_bounds")
            nhv = lax.fori_loop(0, SC_C // (2 * LN), scan2,
                                jnp.zeros((LN,), jnp.int32))
            nh = jnp.max(nhv)

            def sub(j, _):
                j0 = j * SUBC
                for g3 in range(SUBC // LN):
                    ej = plsc.load_gather(hits, [j0 + g3 * LN + iota])
                    plsc.store_scatter(rowbuf, [g3 * LN + iota],
                                       lax.shift_right_logical(ej, 3))
                pltpu.async_copy(rec_hbm.at[rowbuf], recb, sem).wait()
                for g3 in range(SUBC // LN):
                    ej = plsc.load_gather(hits, [j0 + g3 * LN + iota])
                    sub16 = lax.bitwise_and(ej, 7) * 16
                    for t in range(L):
                        ft = plsc.load_gather(
                            recb, [g3 * LN + iota, sub16 + t])
                        plsc.store_scatter(
                            fidx2, [iota * 3 + (g3 * 48 + t)], ft)
                pltpu.async_copy(f_hbm.at[fidx2], frows, sem).wait()

                def comb(e, _):
                    for dc in range(IN_DIM // LN):
                        v = (frows[e * 3, pl.ds(dc * LN, LN)]
                             + frows[e * 3 + 1, pl.ds(dc * LN, LN)]
                             + frows[e * 3 + 2, pl.ds(dc * LN, LN)])
                        plsc.store_scatter(
                            hidt, [(dc * LN + iota) * 97 + e],
                            v * (1.0 / 3.0))
                    return 0
                lax.fori_loop(0, SUBC, comb, 0)

                for g3 in range(SUBC // LN):
                    lanepos = j0 + g3 * LN + iota
                    mk2 = lanepos < nh
                    ej = plsc.load_gather(hits, [j0 + g3 * LN + iota])
                    sub16 = lax.bitwise_and(ej, 7) * 16
                    dv2 = plsc.load_gather(recb, [g3 * LN + iota, sub16 + 3])
                    dstl = jnp.clip(dv2 - nbase, 0, NN - 1)
                    ws = []
                    for h in range(NUM_HEADS):
                        aeh = plsc.bitcast(
                            plsc.load_gather(
                                recb, [g3 * LN + iota, sub16 + 4 + h]),
                            jnp.float32)
                        ash = plsc.load_gather(asumo, [h * NN + dstl])
                        ws.append(aeh / (ash + 1e-16))

                    def dloop(d, _):
                        hd = plsc.load_gather(
                            hidt, [iota + (g3 * LN + d * 97)])
                        for h in range(NUM_HEADS):
                            plsc.addupdate_scatter(
                                acc, [dstl + (h * IN_DIM + d) * NN],
                                ws[h] * hd, mask=mk2)
                        return 0
                    lax.fori_loop(0, IN_DIM, dloop, 0)
                return 0
            lax.fori_loop(0, (nh + SUBC - 1) // SUBC, sub, 0)
            return 0
        lax.fori_loop(0, NSCCH, chunk, 0)

        pltpu.sync_copy(acc, out_hbm.at[pl.ds(nbase * NUM_HEADS * IN_DIM,
                                              ACC_W)])
        return 0
    lax.fori_loop(0, ROUNDS, round_body, 0)


def _make_pass2():
    mesh = plsc.VectorSubcoreMesh(core_axis_name="c", subcore_axis_name="s")
    return pl.kernel(
        _pass2_body,
        out_type=jax.ShapeDtypeStruct((NPAD * NUM_HEADS * IN_DIM,),
                                      jnp.float32),
        mesh=mesh,
        compiler_params=pltpu.CompilerParams(needs_layout_passes=False),
        scratch_types=[
            pltpu.VMEM((SC_C,), jnp.int32),
            pltpu.VMEM((SC_C,), jnp.int32),
            pltpu.VMEM((SUBC,), jnp.int32),
            pltpu.VMEM((SUBC, 128), jnp.int32),
            pltpu.VMEM((SUBC * L,), jnp.int32),
            pltpu.VMEM((SUBC * L, IN_DIM), jnp.float32),
            pltpu.VMEM((12336,), jnp.float32),
            pltpu.VMEM((NN * NUM_HEADS,), jnp.float32),
            pltpu.VMEM((ACC_W,), jnp.float32),
            pltpu.SemaphoreType.DMA,
        ],
    )


def _b1_kernel(ft0_ref, ft1_ref, w_ref, b_ref, o0_ref, o1_ref, p_ref):
    for i, (ft_ref, o_ref) in enumerate([(ft0_ref, o0_ref),
                                         (ft1_ref, o1_ref)]):
        x = ft_ref[...]
        o = jnp.where(x > 0, x, jnp.exp(jnp.minimum(x, 0.0)) - 1.0)
        o_ref[...] = o
        f1 = jnp.tanh(jax.lax.dot_general(
            o, w_ref[...], (((1,), (1,)), ((), ())),
            preferred_element_type=jnp.float32) + b_ref[...][None, :])
        p_ref[0, i] = jnp.sum(f1, axis=0)


def _run_b1(ft0, ft1, fc1_w, fc1_b):
    blk = 1000
    grid = N_NODES // blk
    dh = NUM_HEADS * IN_DIM
    return pl.pallas_call(
        _b1_kernel,
        grid=(grid,),
        in_specs=[
            pl.BlockSpec((blk, dh), lambda i: (i, 0)),
            pl.BlockSpec((blk, dh), lambda i: (i, 0)),
            pl.BlockSpec((128, dh), lambda i: (0, 0)),
            pl.BlockSpec((128,), lambda i: (0,)),
        ],
        out_specs=[
            pl.BlockSpec((blk, dh), lambda i: (i, 0)),
            pl.BlockSpec((blk, dh), lambda i: (i, 0)),
            pl.BlockSpec((1, 2, 128), lambda i: (i, 0, 0)),
        ],
        out_shape=[
            jax.ShapeDtypeStruct((N_NODES, dh), jnp.float32),
            jax.ShapeDtypeStruct((N_NODES, dh), jnp.float32),
            jax.ShapeDtypeStruct((grid, 2, 128), jnp.float32),
        ],
    )(ft0, ft1, fc1_w, fc1_b)


def _final_kernel(o0_ref, o1_ref, beta_ref, fcw_ref, fcb_ref, h_ref, hfc_ref):
    b0 = beta_ref[0, 0]
    b1 = beta_ref[0, 1]
    h = b0 * o0_ref[...] + b1 * o1_ref[...]
    h_ref[...] = h
    hfc_ref[...] = jax.lax.dot_general(
        h, fcw_ref[...], (((1,), (1,)), ((), ())),
        preferred_element_type=jnp.float32) + fcb_ref[...][None, :]


def _run_final(o0, o1, beta, fc_w, fc_b):
    blk = 1000
    grid = N_NODES // blk
    dh = NUM_HEADS * IN_DIM
    h, h_fc = pl.pallas_call(
        _final_kernel,
        grid=(grid,),
        in_specs=[
            pl.BlockSpec((blk, dh), lambda i: (i, 0)),
            pl.BlockSpec((blk, dh), lambda i: (i, 0)),
            pl.BlockSpec((1, 2), lambda i: (0, 0)),
            pl.BlockSpec((OUT_DIM, dh), lambda i: (0, 0)),
            pl.BlockSpec((OUT_DIM,), lambda i: (0,)),
        ],
        out_specs=[
            pl.BlockSpec((blk, dh), lambda i: (i, 0)),
            pl.BlockSpec((blk, OUT_DIM), lambda i: (i, 0)),
        ],
        out_shape=[
            jax.ShapeDtypeStruct((N_NODES, dh), jnp.float32),
            jax.ShapeDtypeStruct((N_NODES, OUT_DIM), jnp.float32),
        ],
    )(o0, o1, beta.reshape(1, 2), fc_w, fc_b)
    return h, h_fc


def _rot_matrices(r_vec):
    rv = r_vec / (jnp.linalg.norm(r_vec, axis=2, keepdims=True) + 1e-12)
    r0re, r0im = rv[0, :, 0], rv[0, :, 1]
    r1re, r1im = rv[1, :, 0], rv[1, :, 1]
    p_re = r0re * r1re - r0im * r1im
    p_im = r0re * r1im + r0im * r1re
    ident = (jnp.ones(IN_DIM // 2), jnp.zeros(IN_DIM // 2))
    eye = jnp.eye(IN_DIM // 2, dtype=jnp.float32)
    mats = []
    for re, im in [(p_re, p_im), (r1re, r1im), (r0re, r0im), ident]:
        r2 = jnp.stack([jnp.stack([re, im]), jnp.stack([-im, re])])
        mats.append(jnp.einsum('kl,abk->kalb', eye, r2).reshape(IN_DIM,
                                                                IN_DIM))
    return jnp.stack(mats)


def kernel(features, type_mask, edge_metapath_indices_0,
           edge_metapath_indices_1, dst_0, dst_1, r_vec, attn_0, attn_1,
           fc1_w, fc1_b, fc2_w, fc_w, fc_b):
    attn_cat = jnp.concatenate([attn_0, attn_1], axis=0)
    m = _rot_matrices(r_vec)
    ms_pad = jnp.pad(jnp.einsum('vij,hj->vih', m, attn_cat),
                     ((0, 0), (0, 0), (0, IN_DIM - 16)))
    f_all, s_all = _build_tables(features, m, ms_pad)
    f_flat = f_all.reshape(4 * N_NODES, IN_DIM)
    s_flat = s_all.reshape(4 * N_NODES, IN_DIM)

    pass1s = [_make_pass1(0, N_NODES, 3 * N_NODES, 0),
              _make_pass1(0, 2 * N_NODES, 3 * N_NODES, 8)]
    pass2 = _make_pass2()
    fts = []
    for mp, (idx, dst) in enumerate([(edge_metapath_indices_0, dst_0),
                                     (edge_metapath_indices_1, dst_1)]):
        idx32 = idx.astype(jnp.int32)
        dst32 = dst.astype(jnp.int32)
        pk = jnp.concatenate(
            [idx32, dst32[:, None],
             jnp.zeros((E_MP, 12), jnp.int32)], axis=1).reshape(RECR, 128)
        rec, parts = pass1s[mp](pk, s_flat)
        asum = parts.reshape(NW, N_NODES * NUM_HEADS).sum(axis=0)
        asum_pad = jnp.pad(asum, (0, (NPAD - N_NODES) * NUM_HEADS))
        asum_blk = asum_pad.reshape(NW * ROUNDS, NN, NUM_HEADS).transpose(
            0, 2, 1).reshape(-1)
        ftflat = pass2(rec.reshape(RECR, 128), dst32, asum_blk, f_flat)
        ft = ftflat.reshape(NW * ROUNDS, NUM_HEADS * IN_DIM, NN).transpose(
            0, 2, 1).reshape(NPAD, NUM_HEADS * IN_DIM)
        fts.append(ft[:N_NODES])

    o0, o1, p = _run_b1(fts[0], fts[1], fc1_w, fc1_b)
    f1m = p.sum(axis=0) * (1.0 / N_NODES)
    betas = f1m @ fc2_w[0]
    beta = jax.nn.softmax(betas)
    h, h_fc = _run_final(o0, o1, beta, fc_w, fc_b)
    return (h_fc, h)
